# baseline (speedup 1.0000x reference)
import functools

import jax
import jax.numpy as jnp
from jax import lax
from jax.experimental import pallas as pl
from jax.experimental.pallas import tpu as pltpu
from jax.experimental.pallas import tpu_sc as plsc

N_NODES = 10000
N_EDGES = 320000
D_IN = 128
D_HID = 64
D_OUT = 40
D_OUT_PAD = 48

NP = 10240
NW = 32
CHUNK = 128
E_PAD = 331776
NCH = E_PAD // (NW * CHUNK)
ROWS_PER_SUB = NP // 16

_DUMMY_DST = N_NODES


def _mm1_body(x_ref, w_ref, b_ref, o_ref):
    o_ref[:, :D_HID] = (
        jnp.dot(x_ref[...], w_ref[...], preferred_element_type=jnp.float32)
        + b_ref[0][None, :]
    )


def _mm1(x_pad, w1t, b1row):
    return pl.pallas_call(
        _mm1_body,
        grid=(NP // 2048,),
        in_specs=[
            pl.BlockSpec((2048, D_IN), lambda i: (i, 0)),
            pl.BlockSpec((D_IN, D_HID), lambda i: (0, 0)),
            pl.BlockSpec((8, D_HID), lambda i: (0, 0)),
        ],
        out_specs=pl.BlockSpec((2048, 128), lambda i: (i, 0)),
        out_shape=jax.ShapeDtypeStruct((NP, 128), jnp.float32),
    )(x_pad, w1t, b1row)


def _make_edge_agg(D):
    mesh = plsc.VectorSubcoreMesh(core_axis_name="c", subcore_axis_name="s")

    @functools.partial(
        pl.kernel,
        mesh=mesh,
        compiler_params=pltpu.CompilerParams(use_tc_tiling_on_sc=False),
        out_type=jax.ShapeDtypeStruct((NP, 128), jnp.float32),
        scratch_types=[
            pltpu.VMEM((NCH, CHUNK), jnp.int32),
            pltpu.VMEM((NCH, CHUNK), jnp.int32),
            pltpu.VMEM((CHUNK, D), jnp.float32),
            pltpu.VMEM((CHUNK, D), jnp.float32),
            pltpu.VMEM((CHUNK, D), jnp.float32),
            pltpu.VMEM_SHARED((NP, D), jnp.float32),
            pltpu.VMEM_SHARED((NP, D), jnp.float32),
            pltpu.SemaphoreType.DMA,
            pltpu.SemaphoreType.DMA,
            pltpu.SemaphoreType.DMA,
            pltpu.SemaphoreType.DMA,
            pltpu.SemaphoreType.DMA,
            pltpu.SemaphoreType.DMA,
        ],
    )
    def k(h_hbm, e_hbm, out_hbm,
          src_v, dst_v, buf_0, buf_1, buf_2, acc, htab,
          gs_0, gs_1, gs_2, ss_0, ss_1, ss_2):
        bufs = (buf_0, buf_1, buf_2)
        gsems = (gs_0, gs_1, gs_2)
        ssems = (ss_0, ss_1, ss_2)
        buf_a, sem_b = buf_0, gs_1
        c = lax.axis_index("c")
        s = lax.axis_index("s")
        w = c * 16 + s
        base = s * ROWS_PER_SUB

        pltpu.async_copy(
            h_hbm.at[pl.ds(base, ROWS_PER_SUB), pl.ds(0, D)],
            htab.at[pl.ds(base, ROWS_PER_SUB)],
            sem_b,
        )

        @pl.loop(0, CHUNK)
        def _(r):
            @pl.loop(0, D, step=16)
            def _(col):
                buf_a[r, pl.ds(col, 16)] = jnp.zeros((16,), jnp.float32)

        @pl.loop(0, ROWS_PER_SUB // CHUNK)
        def _(i):
            pltpu.sync_copy(buf_a, acc.at[pl.ds(base + i * CHUNK, CHUNK)])

        pltpu.sync_copy(e_hbm.at[0, w], src_v)
        pltpu.sync_copy(e_hbm.at[1, w], dst_v)
        pltpu.make_async_copy(
            h_hbm.at[pl.ds(base, ROWS_PER_SUB), pl.ds(0, D)],
            htab.at[pl.ds(base, ROWS_PER_SUB)],
            sem_b,
        ).wait()
        plsc.subcore_barrier()

        NB = 3
        for g in range(NB):
            pltpu.async_copy(htab.at[src_v.at[g]], bufs[g], gsems[g])

        @pl.loop(0, NCH // NB - 1)
        def _(p):
            for g in range(NB):
                j = p * NB + g
                pltpu.make_async_copy(
                    htab.at[src_v.at[j]], bufs[g], gsems[g]).wait()
                pltpu.async_copy(
                    bufs[g], acc.at[dst_v.at[j]], ssems[g], add=True)
            for g in range(NB):
                j = p * NB + g
                pltpu.make_async_copy(
                    bufs[g], acc.at[dst_v.at[j]], ssems[g]).wait()
                pltpu.async_copy(
                    htab.at[src_v.at[j + NB]], bufs[g], gsems[g])

        for g in range(NB):
            j = NCH - NB + g
            pltpu.make_async_copy(
                htab.at[src_v.at[j]], bufs[g], gsems[g]).wait()
            pltpu.async_copy(
                bufs[g], acc.at[dst_v.at[j]], ssems[g], add=True)
        for g in range(NB):
            j = NCH - NB + g
            pltpu.make_async_copy(
                bufs[g], acc.at[dst_v.at[j]], ssems[g]).wait()
        plsc.subcore_barrier()

        col0 = c * D
        pltpu.sync_copy(
            acc.at[pl.ds(base, ROWS_PER_SUB)],
            out_hbm.at[pl.ds(base, ROWS_PER_SUB), pl.ds(col0, D)],
        )

    return k


def _mid_body(p_ref, w_ref, b_ref, o_ref):
    blk = p_ref[...]
    agg = blk[:, :D_HID] + blk[:, D_HID:]
    h = jnp.where(agg > 0, agg, jnp.exp(jnp.minimum(agg, 0.0)) - 1.0)
    o_ref[:, :D_OUT_PAD] = (
        jnp.dot(h, w_ref[...], preferred_element_type=jnp.float32)
        + b_ref[0][None, :]
    )


def _mid(parts, w2t, b2row):
    return pl.pallas_call(
        _mid_body,
        grid=(NP // 2048,),
        in_specs=[
            pl.BlockSpec((2048, 128), lambda i: (i, 0)),
            pl.BlockSpec((D_HID, D_OUT_PAD), lambda i: (0, 0)),
            pl.BlockSpec((8, D_OUT_PAD), lambda i: (0, 0)),
        ],
        out_specs=pl.BlockSpec((2048, 128), lambda i: (i, 0)),
        out_shape=jax.ShapeDtypeStruct((NP, 128), jnp.float32),
    )(parts, w2t, b2row)


def _final_body(p_ref, o_ref):
    blk = p_ref[...]
    logits = blk[:, :D_OUT] + blk[:, D_OUT_PAD:D_OUT_PAD + D_OUT]
    m = jnp.max(logits, axis=1, keepdims=True)
    e = jnp.exp(logits - m)
    lse = jnp.log(jnp.sum(e, axis=1, keepdims=True)) + m
    o_ref[...] = logits - lse


def _final(parts):
    return pl.pallas_call(
        _final_body,
        grid=(NP // 2048,),
        in_specs=[pl.BlockSpec((2048, 128), lambda i: (i, 0))],
        out_specs=pl.BlockSpec((2048, D_OUT), lambda i: (i, 0)),
        out_shape=jax.ShapeDtypeStruct((NP, D_OUT), jnp.float32),
    )(parts)


def kernel(x, edge_index, W1, b1, W2, b2):
    pad = E_PAD - N_EDGES
    dummy = _DUMMY_DST + jnp.arange(pad, dtype=jnp.int32) % (NP - N_NODES)
    pad_block = jnp.stack([jnp.zeros((pad,), jnp.int32), dummy])
    edges = jnp.concatenate([edge_index.astype(jnp.int32), pad_block], axis=1)
    edges = edges.reshape(2, NW, NCH, CHUNK)

    x_pad = jnp.pad(x, ((0, NP - N_NODES), (0, 0)))
    w1t = W1.T
    b1row = jnp.tile(b1[None, :], (8, 1))
    w2t = jnp.pad(W2, ((0, D_OUT_PAD - D_OUT), (0, 0))).T
    b2row = jnp.tile(jnp.pad(b2, (0, D_OUT_PAD - D_OUT))[None, :], (8, 1))

    h1 = _mm1(x_pad, w1t, b1row)
    parts1 = _make_edge_agg(D_HID)(h1, edges)
    h2 = _mid(parts1, w2t, b2row)
    parts2 = _make_edge_agg(D_OUT_PAD)(h2, edges)
    out = _final(parts2)
    return out[:N_NODES]


# speedup vs baseline: 1.2049x; 1.2049x over previous
import functools

import jax
import jax.numpy as jnp
from jax import lax
from jax.experimental import pallas as pl
from jax.experimental.pallas import tpu as pltpu
from jax.experimental.pallas import tpu_sc as plsc

N_NODES = 10000
N_EDGES = 320000
D_IN = 128
D_HID = 64
D_OUT = 40
D_OUT_PAD = 48

NP = 10240
NW = 32
CHUNK = 128
E_PAD = 323584
NCH = E_PAD // (NW * CHUNK)
ROWS_PER_SUB = NP // 16

_DUMMY_DST = N_NODES


def _mm1_body(x_ref, w_ref, b_ref, o_ref):
    o_ref[:, :D_HID] = (
        jnp.dot(x_ref[...], w_ref[...], preferred_element_type=jnp.float32)
        + b_ref[0][None, :]
    )


def _mm1(x_pad, w1t, b1row):
    return pl.pallas_call(
        _mm1_body,
        grid=(NP // 2048,),
        in_specs=[
            pl.BlockSpec((2048, D_IN), lambda i: (i, 0)),
            pl.BlockSpec((D_IN, D_HID), lambda i: (0, 0)),
            pl.BlockSpec((8, D_HID), lambda i: (0, 0)),
        ],
        out_specs=pl.BlockSpec((2048, 128), lambda i: (i, 0)),
        out_shape=jax.ShapeDtypeStruct((NP, 128), jnp.float32),
    )(x_pad, w1t, b1row)


def _make_edge_agg(D):
    mesh = plsc.VectorSubcoreMesh(core_axis_name="c", subcore_axis_name="s")

    @functools.partial(
        pl.kernel,
        mesh=mesh,
        compiler_params=pltpu.CompilerParams(use_tc_tiling_on_sc=False),
        out_type=jax.ShapeDtypeStruct((NP, 128), jnp.float32),
        scratch_types=[
            pltpu.VMEM((NCH, CHUNK), jnp.int32),
            pltpu.VMEM((NCH, CHUNK), jnp.int32),
            pltpu.VMEM((CHUNK, D), jnp.float32),
            pltpu.VMEM((CHUNK, D), jnp.float32),
            pltpu.VMEM_SHARED((NP, D), jnp.float32),
            pltpu.VMEM_SHARED((NP, D), jnp.float32),
            pltpu.SemaphoreType.DMA,
            pltpu.SemaphoreType.DMA,
        ],
    )
    def k(h_hbm, e_hbm, out_hbm,
          src_v, dst_v, buf_a, buf_b, acc, htab, sem_a, sem_b):
        c = lax.axis_index("c")
        s = lax.axis_index("s")
        w = c * 16 + s
        base = s * ROWS_PER_SUB

        pltpu.async_copy(
            h_hbm.at[pl.ds(base, ROWS_PER_SUB), pl.ds(0, D)],
            htab.at[pl.ds(base, ROWS_PER_SUB)],
            sem_b,
        )

        @pl.loop(0, CHUNK)
        def _(r):
            @pl.loop(0, D, step=16)
            def _(col):
                buf_a[r, pl.ds(col, 16)] = jnp.zeros((16,), jnp.float32)

        @pl.loop(0, ROWS_PER_SUB // CHUNK)
        def _(i):
            pltpu.sync_copy(buf_a, acc.at[pl.ds(base + i * CHUNK, CHUNK)])

        pltpu.sync_copy(e_hbm.at[0, w], src_v)
        pltpu.sync_copy(e_hbm.at[1, w], dst_v)
        pltpu.make_async_copy(
            h_hbm.at[pl.ds(base, ROWS_PER_SUB), pl.ds(0, D)],
            htab.at[pl.ds(base, ROWS_PER_SUB)],
            sem_b,
        ).wait()
        plsc.subcore_barrier()

        pltpu.async_copy(htab.at[src_v.at[0]], buf_a, sem_a)

        @pl.loop(0, (NCH - 1) // 2)
        def _(p):
            j = p * 2
            pltpu.async_copy(htab.at[src_v.at[j + 1]], buf_b, sem_b)
            pltpu.make_async_copy(htab.at[src_v.at[j]], buf_a, sem_a).wait()
            pltpu.sync_copy(buf_a, acc.at[dst_v.at[j]], add=True)
            pltpu.async_copy(htab.at[src_v.at[j + 2]], buf_a, sem_a)
            pltpu.make_async_copy(
                htab.at[src_v.at[j + 1]], buf_b, sem_b).wait()
            pltpu.sync_copy(buf_b, acc.at[dst_v.at[j + 1]], add=True)

        pltpu.make_async_copy(
            htab.at[src_v.at[NCH - 1]], buf_a, sem_a).wait()
        pltpu.sync_copy(buf_a, acc.at[dst_v.at[NCH - 1]], add=True)
        plsc.subcore_barrier()

        col0 = c * D
        pltpu.sync_copy(
            acc.at[pl.ds(base, ROWS_PER_SUB)],
            out_hbm.at[pl.ds(base, ROWS_PER_SUB), pl.ds(col0, D)],
        )

    return k


def _mid_body(p_ref, w_ref, b_ref, o_ref):
    blk = p_ref[...]
    agg = blk[:, :D_HID] + blk[:, D_HID:]
    h = jnp.where(agg > 0, agg, jnp.exp(jnp.minimum(agg, 0.0)) - 1.0)
    o_ref[:, :D_OUT_PAD] = (
        jnp.dot(h, w_ref[...], preferred_element_type=jnp.float32)
        + b_ref[0][None, :]
    )


def _mid(parts, w2t, b2row):
    return pl.pallas_call(
        _mid_body,
        grid=(NP // 2048,),
        in_specs=[
            pl.BlockSpec((2048, 128), lambda i: (i, 0)),
            pl.BlockSpec((D_HID, D_OUT_PAD), lambda i: (0, 0)),
            pl.BlockSpec((8, D_OUT_PAD), lambda i: (0, 0)),
        ],
        out_specs=pl.BlockSpec((2048, 128), lambda i: (i, 0)),
        out_shape=jax.ShapeDtypeStruct((NP, 128), jnp.float32),
    )(parts, w2t, b2row)


def _final_body(p_ref, o_ref):
    blk = p_ref[...]
    logits = blk[:, :D_OUT] + blk[:, D_OUT_PAD:D_OUT_PAD + D_OUT]
    m = jnp.max(logits, axis=1, keepdims=True)
    e = jnp.exp(logits - m)
    lse = jnp.log(jnp.sum(e, axis=1, keepdims=True)) + m
    o_ref[...] = logits - lse


def _final(parts):
    return pl.pallas_call(
        _final_body,
        grid=(NP // 2048,),
        in_specs=[pl.BlockSpec((2048, 128), lambda i: (i, 0))],
        out_specs=pl.BlockSpec((2048, D_OUT), lambda i: (i, 0)),
        out_shape=jax.ShapeDtypeStruct((NP, D_OUT), jnp.float32),
    )(parts)


def kernel(x, edge_index, W1, b1, W2, b2):
    pad = E_PAD - N_EDGES
    dummy = _DUMMY_DST + jnp.arange(pad, dtype=jnp.int32) % (NP - N_NODES)
    pad_block = jnp.stack([jnp.zeros((pad,), jnp.int32), dummy])
    edges = jnp.concatenate([edge_index.astype(jnp.int32), pad_block], axis=1)
    edges = edges.reshape(2, NW, NCH, CHUNK)

    x_pad = jnp.pad(x, ((0, NP - N_NODES), (0, 0)))
    w1t = W1.T
    b1row = jnp.tile(b1[None, :], (8, 1))
    w2t = jnp.pad(W2, ((0, D_OUT_PAD - D_OUT), (0, 0))).T
    b2row = jnp.tile(jnp.pad(b2, (0, D_OUT_PAD - D_OUT))[None, :], (8, 1))

    h1 = _mm1(x_pad, w1t, b1row)
    parts1 = _make_edge_agg(D_HID)(h1, edges)
    h2 = _mid(parts1, w2t, b2row)
    parts2 = _make_edge_agg(D_OUT_PAD)(h2, edges)
    out = _final(parts2)
    return out[:N_NODES]


# speedup vs baseline: 1.2087x; 1.0032x over previous
import functools

import jax
import jax.numpy as jnp
from jax import lax
from jax.experimental import pallas as pl
from jax.experimental.pallas import tpu as pltpu
from jax.experimental.pallas import tpu_sc as plsc

N_NODES = 10000
N_EDGES = 320000
D_IN = 128
D_HID = 64
D_OUT = 40
D_OUT_PAD = 48

NP = 10240
NW = 32
CHUNK = 128
E_PAD = 323584
NCH = E_PAD // (NW * CHUNK)
ROWS_PER_SUB = NP // 16

_DUMMY_DST = N_NODES


def _mm1_body(x_ref, w_ref, b_ref, o_ref):
    o_ref[:, :D_HID] = (
        jax.lax.dot_general(
            x_ref[...], w_ref[...], (((1,), (1,)), ((), ())),
            preferred_element_type=jnp.float32)
        + b_ref[0][None, :]
    )


def _mm1(x_pad, w1, b1row):
    return pl.pallas_call(
        _mm1_body,
        grid=(NP // 2048,),
        in_specs=[
            pl.BlockSpec((2048, D_IN), lambda i: (i, 0)),
            pl.BlockSpec((D_HID, D_IN), lambda i: (0, 0)),
            pl.BlockSpec((8, D_HID), lambda i: (0, 0)),
        ],
        out_specs=pl.BlockSpec((2048, 128), lambda i: (i, 0)),
        out_shape=jax.ShapeDtypeStruct((NP, 128), jnp.float32),
    )(x_pad, w1, b1row)


def _make_edge_agg(D):
    mesh = plsc.VectorSubcoreMesh(core_axis_name="c", subcore_axis_name="s")

    @functools.partial(
        pl.kernel,
        mesh=mesh,
        compiler_params=pltpu.CompilerParams(use_tc_tiling_on_sc=False),
        out_type=jax.ShapeDtypeStruct((NP, 128), jnp.float32),
        scratch_types=[
            pltpu.VMEM((NCH, CHUNK), jnp.int32),
            pltpu.VMEM((NCH, CHUNK), jnp.int32),
            pltpu.VMEM((CHUNK, D), jnp.float32),
            pltpu.VMEM((CHUNK, D), jnp.float32),
            pltpu.VMEM_SHARED((NP, D), jnp.float32),
            pltpu.VMEM_SHARED((NP, D), jnp.float32),
            pltpu.SemaphoreType.DMA,
            pltpu.SemaphoreType.DMA,
        ],
    )
    def k(h_hbm, e_hbm, out_hbm,
          src_v, dst_v, buf_a, buf_b, acc, htab, sem_a, sem_b):
        c = lax.axis_index("c")
        s = lax.axis_index("s")
        w = c * 16 + s
        base = s * ROWS_PER_SUB

        pltpu.async_copy(
            h_hbm.at[pl.ds(base, ROWS_PER_SUB), pl.ds(0, D)],
            htab.at[pl.ds(base, ROWS_PER_SUB)],
            sem_b,
        )

        @pl.loop(0, CHUNK)
        def _(r):
            @pl.loop(0, D, step=16)
            def _(col):
                buf_a[r, pl.ds(col, 16)] = jnp.zeros((16,), jnp.float32)

        @pl.loop(0, ROWS_PER_SUB // CHUNK)
        def _(i):
            pltpu.sync_copy(buf_a, acc.at[pl.ds(base + i * CHUNK, CHUNK)])

        pltpu.sync_copy(e_hbm.at[0, w], src_v)
        pltpu.sync_copy(e_hbm.at[1, w], dst_v)
        pltpu.make_async_copy(
            h_hbm.at[pl.ds(base, ROWS_PER_SUB), pl.ds(0, D)],
            htab.at[pl.ds(base, ROWS_PER_SUB)],
            sem_b,
        ).wait()
        plsc.subcore_barrier()

        pltpu.async_copy(htab.at[src_v.at[0]], buf_a, sem_a)

        @pl.loop(0, (NCH - 1) // 2)
        def _(p):
            j = p * 2
            pltpu.async_copy(htab.at[src_v.at[j + 1]], buf_b, sem_b)
            pltpu.make_async_copy(htab.at[src_v.at[j]], buf_a, sem_a).wait()
            pltpu.sync_copy(buf_a, acc.at[dst_v.at[j]], add=True)
            pltpu.async_copy(htab.at[src_v.at[j + 2]], buf_a, sem_a)
            pltpu.make_async_copy(
                htab.at[src_v.at[j + 1]], buf_b, sem_b).wait()
            pltpu.sync_copy(buf_b, acc.at[dst_v.at[j + 1]], add=True)

        pltpu.make_async_copy(
            htab.at[src_v.at[NCH - 1]], buf_a, sem_a).wait()
        pltpu.sync_copy(buf_a, acc.at[dst_v.at[NCH - 1]], add=True)
        plsc.subcore_barrier()

        col0 = c * D
        pltpu.sync_copy(
            acc.at[pl.ds(base, ROWS_PER_SUB)],
            out_hbm.at[pl.ds(base, ROWS_PER_SUB), pl.ds(col0, D)],
        )

    return k


def _mid_body(p_ref, w_ref, b_ref, o_ref):
    blk = p_ref[...]
    agg = blk[:, :D_HID] + blk[:, D_HID:]
    h = jnp.where(agg > 0, agg, jnp.exp(jnp.minimum(agg, 0.0)) - 1.0)
    o_ref[:, :D_OUT_PAD] = (
        jax.lax.dot_general(
            h, w_ref[...], (((1,), (1,)), ((), ())),
            preferred_element_type=jnp.float32)
        + b_ref[0][None, :]
    )


def _mid(parts, w2p, b2row):
    return pl.pallas_call(
        _mid_body,
        grid=(NP // 2048,),
        in_specs=[
            pl.BlockSpec((2048, 128), lambda i: (i, 0)),
            pl.BlockSpec((D_OUT_PAD, D_HID), lambda i: (0, 0)),
            pl.BlockSpec((8, D_OUT_PAD), lambda i: (0, 0)),
        ],
        out_specs=pl.BlockSpec((2048, 128), lambda i: (i, 0)),
        out_shape=jax.ShapeDtypeStruct((NP, 128), jnp.float32),
    )(parts, w2p, b2row)


def _final_body(p_ref, o_ref):
    blk = p_ref[...]
    logits = blk[:, :D_OUT] + blk[:, D_OUT_PAD:D_OUT_PAD + D_OUT]
    m = jnp.max(logits, axis=1, keepdims=True)
    e = jnp.exp(logits - m)
    lse = jnp.log(jnp.sum(e, axis=1, keepdims=True)) + m
    o_ref[...] = logits - lse


def _final(parts):
    return pl.pallas_call(
        _final_body,
        grid=(N_NODES // 2000,),
        in_specs=[pl.BlockSpec((2000, 128), lambda i: (i, 0))],
        out_specs=pl.BlockSpec((2000, D_OUT), lambda i: (i, 0)),
        out_shape=jax.ShapeDtypeStruct((N_NODES, D_OUT), jnp.float32),
    )(parts)


def kernel(x, edge_index, W1, b1, W2, b2):
    pad = E_PAD - N_EDGES
    dummy = _DUMMY_DST + jnp.arange(pad, dtype=jnp.int32) % (NP - N_NODES)
    pad_block = jnp.stack([jnp.zeros((pad,), jnp.int32), dummy])
    edges = jnp.concatenate([edge_index.astype(jnp.int32), pad_block], axis=1)
    edges = edges.reshape(2, NW, NCH, CHUNK)

    x_pad = jnp.pad(x, ((0, NP - N_NODES), (0, 0)))
    b1row = jnp.tile(b1[None, :], (8, 1))
    w2p = jnp.pad(W2, ((0, D_OUT_PAD - D_OUT), (0, 0)))
    b2row = jnp.tile(jnp.pad(b2, (0, D_OUT_PAD - D_OUT))[None, :], (8, 1))

    h1 = _mm1(x_pad, W1, b1row)
    parts1 = _make_edge_agg(D_HID)(h1, edges)
    h2 = _mid(parts1, w2p, b2row)
    parts2 = _make_edge_agg(D_OUT_PAD)(h2, edges)
    return _final(parts2)


# speedup vs baseline: 1.2500x; 1.0342x over previous
import functools

import jax
import jax.numpy as jnp
from jax import lax
from jax.experimental import pallas as pl
from jax.experimental.pallas import tpu as pltpu
from jax.experimental.pallas import tpu_sc as plsc

N_NODES = 10000
N_EDGES = 320000
D_IN = 128
D_HID = 64
D_OUT = 40
D_OUT_PAD = 48

NP = 10240
NW = 32
CHUNK = 128
E_PAD = 323584
NCH = E_PAD // (NW * CHUNK)
ROWS_PER_SUB = NP // 16

_DUMMY_DST = N_NODES


def _mm1_body(x_ref, w_ref, b_ref, o_ref):
    o_ref[:N_NODES, :D_HID] = (
        jax.lax.dot_general(
            x_ref[...], w_ref[...], (((1,), (1,)), ((), ())),
            preferred_element_type=jnp.float32)
        + b_ref[0][None, :]
    )


def _mm1(x, w1, b1row):
    return pl.pallas_call(
        _mm1_body,
        grid=(1,),
        in_specs=[
            pl.BlockSpec((N_NODES, D_IN), lambda i: (0, 0)),
            pl.BlockSpec((D_HID, D_IN), lambda i: (0, 0)),
            pl.BlockSpec((8, D_HID), lambda i: (0, 0)),
        ],
        out_specs=pl.BlockSpec((NP, 128), lambda i: (0, 0)),
        out_shape=jax.ShapeDtypeStruct((NP, 128), jnp.float32),
    )(x, w1, b1row)


def _make_edge_agg(D):
    mesh = plsc.VectorSubcoreMesh(core_axis_name="c", subcore_axis_name="s")

    @functools.partial(
        pl.kernel,
        mesh=mesh,
        compiler_params=pltpu.CompilerParams(use_tc_tiling_on_sc=False),
        out_type=jax.ShapeDtypeStruct((NP, 128), jnp.float32),
        scratch_types=[
            pltpu.VMEM((NCH, CHUNK), jnp.int32),
            pltpu.VMEM((NCH, CHUNK), jnp.int32),
            pltpu.VMEM((CHUNK, D), jnp.float32),
            pltpu.VMEM((CHUNK, D), jnp.float32),
            pltpu.VMEM_SHARED((NP, D), jnp.float32),
            pltpu.VMEM_SHARED((NP, D), jnp.float32),
            pltpu.SemaphoreType.DMA,
            pltpu.SemaphoreType.DMA,
        ],
    )
    def k(h_hbm, e_hbm, out_hbm,
          src_v, dst_v, buf_a, buf_b, acc, htab, sem_a, sem_b):
        c = lax.axis_index("c")
        s = lax.axis_index("s")
        w = c * 16 + s
        base = s * ROWS_PER_SUB

        pltpu.async_copy(
            h_hbm.at[pl.ds(base, ROWS_PER_SUB), pl.ds(0, D)],
            htab.at[pl.ds(base, ROWS_PER_SUB)],
            sem_b,
        )

        @pl.loop(0, CHUNK)
        def _(r):
            @pl.loop(0, D, step=16)
            def _(col):
                buf_a[r, pl.ds(col, 16)] = jnp.zeros((16,), jnp.float32)

        @pl.loop(0, ROWS_PER_SUB // CHUNK)
        def _(i):
            pltpu.sync_copy(buf_a, acc.at[pl.ds(base + i * CHUNK, CHUNK)])

        pltpu.sync_copy(e_hbm.at[0, w], src_v)
        pltpu.sync_copy(e_hbm.at[1, w], dst_v)
        pltpu.make_async_copy(
            h_hbm.at[pl.ds(base, ROWS_PER_SUB), pl.ds(0, D)],
            htab.at[pl.ds(base, ROWS_PER_SUB)],
            sem_b,
        ).wait()
        plsc.subcore_barrier()

        pltpu.async_copy(htab.at[src_v.at[0]], buf_a, sem_a)

        @pl.loop(0, (NCH - 1) // 2)
        def _(p):
            j = p * 2
            pltpu.async_copy(htab.at[src_v.at[j + 1]], buf_b, sem_b)
            pltpu.make_async_copy(htab.at[src_v.at[j]], buf_a, sem_a).wait()
            pltpu.sync_copy(buf_a, acc.at[dst_v.at[j]], add=True)
            pltpu.async_copy(htab.at[src_v.at[j + 2]], buf_a, sem_a)
            pltpu.make_async_copy(
                htab.at[src_v.at[j + 1]], buf_b, sem_b).wait()
            pltpu.sync_copy(buf_b, acc.at[dst_v.at[j + 1]], add=True)

        pltpu.make_async_copy(
            htab.at[src_v.at[NCH - 1]], buf_a, sem_a).wait()
        pltpu.sync_copy(buf_a, acc.at[dst_v.at[NCH - 1]], add=True)
        plsc.subcore_barrier()

        col0 = c * D
        pltpu.sync_copy(
            acc.at[pl.ds(base, ROWS_PER_SUB)],
            out_hbm.at[pl.ds(base, ROWS_PER_SUB), pl.ds(col0, D)],
        )

    return k


def _mid_body(p_ref, w_ref, b_ref, o_ref):
    blk = p_ref[...]
    agg = blk[:, :D_HID] + blk[:, D_HID:]
    h = jnp.where(agg > 0, agg, jnp.exp(jnp.minimum(agg, 0.0)) - 1.0)
    o_ref[:, :D_OUT_PAD] = (
        jax.lax.dot_general(
            h, w_ref[...], (((1,), (1,)), ((), ())),
            preferred_element_type=jnp.float32)
        + b_ref[0][None, :]
    )


def _mid(parts, w2p, b2row):
    return pl.pallas_call(
        _mid_body,
        grid=(1,),
        in_specs=[
            pl.BlockSpec((NP, 128), lambda i: (0, 0)),
            pl.BlockSpec((D_OUT_PAD, D_HID), lambda i: (0, 0)),
            pl.BlockSpec((8, D_OUT_PAD), lambda i: (0, 0)),
        ],
        out_specs=pl.BlockSpec((NP, 128), lambda i: (0, 0)),
        out_shape=jax.ShapeDtypeStruct((NP, 128), jnp.float32),
    )(parts, w2p, b2row)


def _final_body(p_ref, o_ref):
    blk = p_ref[...]
    logits = blk[:, :D_OUT] + blk[:, D_OUT_PAD:D_OUT_PAD + D_OUT]
    m = jnp.max(logits, axis=1, keepdims=True)
    e = jnp.exp(logits - m)
    lse = jnp.log(jnp.sum(e, axis=1, keepdims=True)) + m
    o_ref[...] = logits - lse


def _final(parts):
    return pl.pallas_call(
        _final_body,
        grid=(1,),
        in_specs=[pl.BlockSpec((N_NODES, 128), lambda i: (0, 0))],
        out_specs=pl.BlockSpec((N_NODES, D_OUT), lambda i: (0, 0)),
        out_shape=jax.ShapeDtypeStruct((N_NODES, D_OUT), jnp.float32),
    )(parts)


def kernel(x, edge_index, W1, b1, W2, b2):
    pad = E_PAD - N_EDGES
    dummy = _DUMMY_DST + jnp.arange(pad, dtype=jnp.int32) % (NP - N_NODES)
    pad_block = jnp.stack([jnp.zeros((pad,), jnp.int32), dummy])
    edges = jnp.concatenate([edge_index.astype(jnp.int32), pad_block], axis=1)
    edges = edges.reshape(2, NW, NCH, CHUNK)

    b1row = jnp.tile(b1[None, :], (8, 1))
    w2p = jnp.pad(W2, ((0, D_OUT_PAD - D_OUT), (0, 0)))
    b2row = jnp.tile(jnp.pad(b2, (0, D_OUT_PAD - D_OUT))[None, :], (8, 1))

    h1 = _mm1(x, W1, b1row)
    parts1 = _make_edge_agg(D_HID)(h1, edges)
    h2 = _mid(parts1, w2p, b2row)
    parts2 = _make_edge_agg(D_OUT_PAD)(h2, edges)
    return _final(parts2)
